# baseline (speedup 1.0000x reference)
import functools

import jax
import jax.numpy as jnp
from jax import lax
from jax.experimental import pallas as pl
from jax.experimental.pallas import tpu as pltpu
from jax.experimental.pallas import tpu_sc as plsc

N = 8388608
G = 8192
K = 512
D = 256
EPS = 1e-08

NC = 2
NS = 16
NW = NC * NS
PER_W = N // NW
CH = 16384
NCH = PER_W // CH
L = 16


def _sc_energy_body(h_hbm, g_hbm, out_hbm, hbuf, gbuf, acc, hsem, gsem):
    c = lax.axis_index("c")
    s = lax.axis_index("s")
    wid = s * NC + c
    base = wid * PER_W

    @plsc.parallel_loop(0, G // L, unroll=8)
    def _zero(i):
        acc[pl.ds(pl.multiple_of(i * L, L), L)] = jnp.zeros((L,), jnp.float32)

    def start(k, b):
        off = base + k * CH
        ch = pltpu.async_copy(h_hbm.at[pl.ds(off, CH)], hbuf.at[b], hsem.at[b])
        cg = pltpu.async_copy(g_hbm.at[pl.ds(off, CH)], gbuf.at[b], gsem.at[b])
        return ch, cg

    pend = start(0, 0)
    for k in range(NCH):
        b = k % 2
        cur = pend
        if k + 1 < NCH:
            pend = start(k + 1, 1 - b)
        cur[0].wait()
        cur[1].wait()

        @plsc.parallel_loop(0, CH // L, unroll=16)
        def _scat(i):
            o = pl.ds(pl.multiple_of(i * L, L), L)
            idx = gbuf[b, o]
            x = hbuf[b, o]
            plsc.addupdate_scatter(acc, [idx], x * x)

    pltpu.sync_copy(acc, out_hbm.at[pl.ds(wid * G, G)])


@jax.jit
def _sc_energy(h0, gid):
    mesh = plsc.VectorSubcoreMesh(core_axis_name="c", subcore_axis_name="s")
    return pl.kernel(
        _sc_energy_body,
        out_type=jax.ShapeDtypeStruct((NW * G,), jnp.float32),
        mesh=mesh,
        compiler_params=pltpu.CompilerParams(needs_layout_passes=False),
        scratch_types=[
            pltpu.VMEM((2, CH), jnp.float32),
            pltpu.VMEM((2, CH), jnp.int32),
            pltpu.VMEM((G,), jnp.float32),
            pltpu.SemaphoreType.DMA((2,)),
            pltpu.SemaphoreType.DMA((2,)),
        ],
    )(h0, gid)


def _gelu(x):
    return 0.5 * x * (1.0 + lax.erf(x * 0.7071067811865476))


def _tc_body(part, uv, emb_bf, feat4,
             eW1, eb1, eW2, eb2, pW1, pb1, pW2, pb2, gW1, gb1, gW2, gb2,
             out, klist_ref):
    p = part[...]
    eg = p[0:64, :]
    for w in range(1, NW):
        eg = eg + p[w * 64:(w + 1) * 64, :]
    hnorm2 = jnp.sum(eg)

    row = lax.broadcasted_iota(jnp.int32, (64, 128), 0)
    col = lax.broadcasted_iota(jnp.int32, (64, 128), 1)
    flat = row * 128 + col

    kb = lax.bitcast_convert_type(eg, jnp.int32)
    keys = jnp.bitwise_or(jnp.bitwise_and(kb, jnp.int32(~0x1FFF)),
                          jnp.int32(8191) - flat)
    KMIN = jnp.int32(-(2 ** 31))

    NCHAIN = 4
    RPC = 64 // NCHAIN

    def step(t, carry):
        ms, new = [], []
        for ch in range(NCHAIN):
            kc, cm = carry[ch]
            m = jnp.max(cm, axis=1, keepdims=True)
            kc = jnp.where(kc == m, KMIN, kc)
            cm = jnp.max(kc, axis=0, keepdims=True)
            ms.append(m)
            new.append((kc, cm))
        klist_ref[pl.ds(t, 1), :] = jnp.concatenate(ms, axis=1)
        return tuple(new)

    carry0 = tuple(
        (keys[RPC * c:RPC * (c + 1), :],
         jnp.max(keys[RPC * c:RPC * (c + 1), :], axis=0, keepdims=True))
        for c in range(NCHAIN))
    lax.fori_loop(0, K, step, carry0)

    kl = klist_ref[...]
    vals4 = lax.bitcast_convert_type(
        jnp.bitwise_and(kl, jnp.int32(~0x1FFF)), jnp.float32)
    idx4 = jnp.int32(8191) - jnp.bitwise_and(kl, jnp.int32(0x1FFF))
    klT = lax.transpose(kl, (1, 0))
    pos = lax.broadcasted_iota(jnp.int32, (K, 1), 0)
    lane512 = lax.broadcasted_iota(jnp.int32, (1, K), 1)
    merged = None
    for c in range(NCHAIN):
        colc = kl[:, c:c + 1]
        rank = pos
        for o in range(NCHAIN):
            if o == c:
                continue
            gt = klT[o:o + 1, :] > colc
            rank = rank + jnp.sum(jnp.where(gt, 1, 0), axis=1,
                                  keepdims=True)
        S_T = jnp.where(rank == lane512, 1.0, 0.0)
        cat = jnp.concatenate(
            [vals4[:, c:c + 1], idx4[:, c:c + 1].astype(jnp.float32)],
            axis=1)
        contrib = lax.dot_general(
            S_T, cat, (((0,), (0,)), ((), ())),
            precision=lax.Precision.HIGHEST,
            preferred_element_type=jnp.float32)
        merged = contrib if merged is None else merged + contrib
    vals = merged[:, 0:1]
    idxc = merged[:, 1:2].astype(jnp.int32)

    lane = lax.broadcasted_iota(jnp.int32, (1, G), 1)
    P = jnp.where(idxc == lane, 1.0, 0.0).astype(jnp.bfloat16)
    gid_emb = jnp.dot(P, emb_bf[...], preferred_element_type=jnp.float32)
    uvg = jnp.dot(P, uv[...].astype(jnp.bfloat16),
                  preferred_element_type=jnp.float32)

    mm = functools.partial(jnp.dot, precision=lax.Precision.HIGHEST,
                           preferred_element_type=jnp.float32)
    x = jnp.log(vals + EPS)
    h1 = _gelu(mm(x, eW1[...]) + eb1[...].reshape(1, D))
    e_emb = mm(h1, eW2[...]) + eb2[...].reshape(1, D)
    h2 = _gelu(mm(uvg, pW1[...]) + pb1[...].reshape(1, D))
    p_emb = mm(h2, pW2[...]) + pb2[...].reshape(1, D)
    out[pl.ds(1, K), :] = gid_emb + e_emb + p_emb

    g_pre = (mm(feat4[...], gW1[...][0:4, :])
             + hnorm2 * gW1[...][4:5, :]
             + gb1[...].reshape(1, D))
    out[pl.ds(0, 1), :] = mm(_gelu(g_pre), gW2[...]) + gb2[...].reshape(1, D)


def _tc_tokens(part2d, uv, emb_bf, feat4, *weights):
    return pl.pallas_call(
        _tc_body,
        out_shape=jax.ShapeDtypeStruct((K + 1, D), jnp.float32),
        scratch_shapes=[
            pltpu.VMEM((K, 4), jnp.int32),
        ],
    )(part2d, uv, emb_bf, feat4, *weights)


def kernel(h0, group_id, group_uv, snr_db, P, NrRF, NtRF, group_embed,
           eW1, eb1, eW2, eb2, pW1, pb1, pW2, pb2, gW1, gb1, gW2, gb2):
    part = _sc_energy(h0, group_id.astype(jnp.int32))
    part2d = part.reshape(NW * 64, 128)
    feat4 = jnp.stack([
        jnp.asarray(snr_db, jnp.float32),
        jnp.asarray(P, jnp.float32),
        jnp.asarray(NrRF, jnp.float32),
        jnp.asarray(NtRF, jnp.float32),
    ]).reshape(1, 4)
    return _tc_tokens(part2d, group_uv.astype(jnp.float32),
                      group_embed.astype(jnp.bfloat16), feat4,
                      eW1, eb1, eW2, eb2, pW1, pb1, pW2, pb2,
                      gW1, gb1, gW2, gb2)


# speedup vs baseline: 1.2759x; 1.2759x over previous
import functools

import jax
import jax.numpy as jnp
from jax import lax
from jax.experimental import pallas as pl
from jax.experimental.pallas import tpu as pltpu
from jax.experimental.pallas import tpu_sc as plsc

N = 8388608
G = 8192
K = 512
D = 256
EPS = 1e-08

NC = 2
NS = 16
NW = NC * NS
PER_W = N // NW
CH = 16384
NCH = PER_W // CH
L = 16


def _sc_energy_body(h_hbm, g_hbm, out_hbm, hbuf, gbuf, acc, hsem, gsem):
    c = lax.axis_index("c")
    s = lax.axis_index("s")
    wid = s * NC + c
    base = wid * PER_W

    @plsc.parallel_loop(0, G // L, unroll=8)
    def _zero(i):
        acc[pl.ds(pl.multiple_of(i * L, L), L)] = jnp.zeros((L,), jnp.float32)

    def start(k, b):
        off = base + k * CH
        ch = pltpu.async_copy(h_hbm.at[pl.ds(off, CH)], hbuf.at[b], hsem.at[b])
        cg = pltpu.async_copy(g_hbm.at[pl.ds(off, CH)], gbuf.at[b], gsem.at[b])
        return ch, cg

    pend = start(0, 0)
    for k in range(NCH):
        b = k % 2
        cur = pend
        if k + 1 < NCH:
            pend = start(k + 1, 1 - b)
        cur[0].wait()
        cur[1].wait()

        @plsc.parallel_loop(0, CH // L, unroll=16)
        def _scat(i):
            o = pl.ds(pl.multiple_of(i * L, L), L)
            idx = gbuf[b, o]
            x = hbuf[b, o]
            plsc.addupdate_scatter(acc, [idx], x * x)

    pltpu.sync_copy(acc, out_hbm.at[pl.ds(wid * G, G)])


@jax.jit
def _sc_energy(h0, gid):
    mesh = plsc.VectorSubcoreMesh(core_axis_name="c", subcore_axis_name="s")
    return pl.kernel(
        _sc_energy_body,
        out_type=jax.ShapeDtypeStruct((NW * G,), jnp.float32),
        mesh=mesh,
        compiler_params=pltpu.CompilerParams(needs_layout_passes=False),
        scratch_types=[
            pltpu.VMEM((2, CH), jnp.float32),
            pltpu.VMEM((2, CH), jnp.int32),
            pltpu.VMEM((G,), jnp.float32),
            pltpu.SemaphoreType.DMA((2,)),
            pltpu.SemaphoreType.DMA((2,)),
        ],
    )(h0, gid)


def _gelu(x):
    return 0.5 * x * (1.0 + lax.erf(x * 0.7071067811865476))


def _tc_body(part, uv, emb_bf, feat4,
             eW1, eb1, eW2, eb2, pW1, pb1, pW2, pb2, gW1, gb1, gW2, gb2,
             out, klist_ref):
    p = part[...]
    eg = p[0:64, :]
    for w in range(1, NW):
        eg = eg + p[w * 64:(w + 1) * 64, :]
    hnorm2 = jnp.sum(eg)

    row = lax.broadcasted_iota(jnp.int32, (64, 128), 0)
    col = lax.broadcasted_iota(jnp.int32, (64, 128), 1)
    flat = row * 128 + col

    kb = lax.bitcast_convert_type(eg, jnp.int32)
    keys = lax.bitcast_convert_type(
        jnp.bitwise_or(jnp.bitwise_and(kb, jnp.int32(~0x1FFF)),
                       jnp.int32(8191) - flat), jnp.float32)
    KMIN = jnp.float32(-1.0)

    NCHAIN = 4
    RPC = 64 // NCHAIN

    def step(t, carry):
        ms, new = [], []
        for ch in range(NCHAIN):
            kc, cm = carry[ch]
            m = jnp.max(cm, axis=1, keepdims=True)
            kc = jnp.where(kc == m, KMIN, kc)
            cm = jnp.max(kc, axis=0, keepdims=True)
            ms.append(m)
            new.append((kc, cm))
        klist_ref[pl.ds(t, 1), :] = jnp.concatenate(ms, axis=1)
        return tuple(new)

    carry0 = tuple(
        (keys[RPC * c:RPC * (c + 1), :],
         jnp.max(keys[RPC * c:RPC * (c + 1), :], axis=0, keepdims=True))
        for c in range(NCHAIN))
    lax.fori_loop(0, K, step, carry0)

    kl = klist_ref[...]
    kli = lax.bitcast_convert_type(kl, jnp.int32)
    vals4 = lax.bitcast_convert_type(
        jnp.bitwise_and(kli, jnp.int32(~0x1FFF)), jnp.float32)
    idx4 = jnp.int32(8191) - jnp.bitwise_and(kli, jnp.int32(0x1FFF))
    klT = lax.transpose(kl, (1, 0))
    pos = lax.broadcasted_iota(jnp.int32, (K, 1), 0)
    lane512 = lax.broadcasted_iota(jnp.int32, (1, K), 1)
    merged = None
    for c in range(NCHAIN):
        colc = kl[:, c:c + 1]
        rank = pos
        for o in range(NCHAIN):
            if o == c:
                continue
            gt = klT[o:o + 1, :] > colc
            rank = rank + jnp.sum(jnp.where(gt, 1, 0), axis=1,
                                  keepdims=True)
        S_T = jnp.where(rank == lane512, 1.0, 0.0)
        cat = jnp.concatenate(
            [vals4[:, c:c + 1], idx4[:, c:c + 1].astype(jnp.float32)],
            axis=1)
        contrib = lax.dot_general(
            S_T, cat, (((0,), (0,)), ((), ())),
            precision=lax.Precision.HIGHEST,
            preferred_element_type=jnp.float32)
        merged = contrib if merged is None else merged + contrib
    vals = merged[:, 0:1]
    idxc = merged[:, 1:2].astype(jnp.int32)

    lane = lax.broadcasted_iota(jnp.int32, (1, G), 1)
    P = jnp.where(idxc == lane, 1.0, 0.0).astype(jnp.bfloat16)
    gid_emb = jnp.dot(P, emb_bf[...], preferred_element_type=jnp.float32)
    uvg = jnp.dot(P, uv[...].astype(jnp.bfloat16),
                  preferred_element_type=jnp.float32)

    mm = functools.partial(jnp.dot, precision=lax.Precision.HIGHEST,
                           preferred_element_type=jnp.float32)
    x = jnp.log(vals + EPS)
    h1 = _gelu(mm(x, eW1[...]) + eb1[...].reshape(1, D))
    e_emb = mm(h1, eW2[...]) + eb2[...].reshape(1, D)
    h2 = _gelu(mm(uvg, pW1[...]) + pb1[...].reshape(1, D))
    p_emb = mm(h2, pW2[...]) + pb2[...].reshape(1, D)
    out[pl.ds(1, K), :] = gid_emb + e_emb + p_emb

    g_pre = (mm(feat4[...], gW1[...][0:4, :])
             + hnorm2 * gW1[...][4:5, :]
             + gb1[...].reshape(1, D))
    out[pl.ds(0, 1), :] = mm(_gelu(g_pre), gW2[...]) + gb2[...].reshape(1, D)


def _tc_tokens(part2d, uv, emb_bf, feat4, *weights):
    return pl.pallas_call(
        _tc_body,
        out_shape=jax.ShapeDtypeStruct((K + 1, D), jnp.float32),
        scratch_shapes=[
            pltpu.VMEM((K, 4), jnp.float32),
        ],
    )(part2d, uv, emb_bf, feat4, *weights)


def kernel(h0, group_id, group_uv, snr_db, P, NrRF, NtRF, group_embed,
           eW1, eb1, eW2, eb2, pW1, pb1, pW2, pb2, gW1, gb1, gW2, gb2):
    part = _sc_energy(h0, group_id.astype(jnp.int32))
    part2d = part.reshape(NW * 64, 128)
    feat4 = jnp.stack([
        jnp.asarray(snr_db, jnp.float32),
        jnp.asarray(P, jnp.float32),
        jnp.asarray(NrRF, jnp.float32),
        jnp.asarray(NtRF, jnp.float32),
    ]).reshape(1, 4)
    return _tc_tokens(part2d, group_uv.astype(jnp.float32),
                      group_embed.astype(jnp.bfloat16), feat4,
                      eW1, eb1, eW2, eb2, pW1, pb1, pW2, pb2,
                      gW1, gb1, gW2, gb2)


# speedup vs baseline: 1.5676x; 1.2286x over previous
import functools

import jax
import jax.numpy as jnp
from jax import lax
from jax.experimental import pallas as pl
from jax.experimental.pallas import tpu as pltpu
from jax.experimental.pallas import tpu_sc as plsc

N = 8388608
G = 8192
K = 512
D = 256
EPS = 1e-08

NC = 2
NS = 16
NW = NC * NS
PER_W = N // NW
CH = 16384
NCH = PER_W // CH
L = 16


def _sc_energy_body(h_hbm, g_hbm, out_hbm, hbuf, gbuf, acc, hsem, gsem):
    c = lax.axis_index("c")
    s = lax.axis_index("s")
    wid = s * NC + c
    base = wid * PER_W

    @plsc.parallel_loop(0, G // L, unroll=8)
    def _zero(i):
        acc[pl.ds(pl.multiple_of(i * L, L), L)] = jnp.zeros((L,), jnp.float32)

    def start(k, b):
        off = base + k * CH
        ch = pltpu.async_copy(h_hbm.at[pl.ds(off, CH)], hbuf.at[b], hsem.at[b])
        cg = pltpu.async_copy(g_hbm.at[pl.ds(off, CH)], gbuf.at[b], gsem.at[b])
        return ch, cg

    pend = start(0, 0)
    for k in range(NCH):
        b = k % 2
        cur = pend
        if k + 1 < NCH:
            pend = start(k + 1, 1 - b)
        cur[0].wait()
        cur[1].wait()

        @plsc.parallel_loop(0, CH // L, unroll=16)
        def _scat(i):
            o = pl.ds(pl.multiple_of(i * L, L), L)
            idx = gbuf[b, o]
            x = hbuf[b, o]
            plsc.addupdate_scatter(acc, [idx], x * x)

    pltpu.sync_copy(acc, out_hbm.at[pl.ds(wid * G, G)])


@jax.jit
def _sc_energy(h0, gid):
    mesh = plsc.VectorSubcoreMesh(core_axis_name="c", subcore_axis_name="s")
    return pl.kernel(
        _sc_energy_body,
        out_type=jax.ShapeDtypeStruct((NW * G,), jnp.float32),
        mesh=mesh,
        compiler_params=pltpu.CompilerParams(needs_layout_passes=False),
        scratch_types=[
            pltpu.VMEM((2, CH), jnp.float32),
            pltpu.VMEM((2, CH), jnp.int32),
            pltpu.VMEM((G,), jnp.float32),
            pltpu.SemaphoreType.DMA((2,)),
            pltpu.SemaphoreType.DMA((2,)),
        ],
    )(h0, gid)


def _gelu(x):
    return 0.5 * x * (1.0 + lax.erf(x * 0.7071067811865476))


def _tc_body(part, uv, emb_bf, feat4,
             eW1, eb1, eW2, eb2, pW1, pb1, pW2, pb2, gW1, gb1, gW2, gb2,
             out, klist_ref):
    p = part[...]
    eg = p[0:64, :]
    for w in range(1, NW):
        eg = eg + p[w * 64:(w + 1) * 64, :]
    hnorm2 = jnp.sum(eg)

    row = lax.broadcasted_iota(jnp.int32, (64, 128), 0)
    col = lax.broadcasted_iota(jnp.int32, (64, 128), 1)
    flat = row * 128 + col

    kb = lax.bitcast_convert_type(eg, jnp.int32)
    ki = jnp.bitwise_or(jnp.bitwise_and(kb, jnp.int32(~0x1FFF)),
                        jnp.int32(8191) - flat)
    keys = lax.bitcast_convert_type(ki, jnp.float32)
    KMIN = jnp.float32(-1.0)

    NCHAIN = 4
    RPC = 64 // NCHAIN

    klist_ref[...] = jnp.zeros((K, NCHAIN), jnp.float32)

    def bs_step(i, t):
        cand = jnp.bitwise_or(t, lax.shift_left(jnp.int32(1),
                                                jnp.int32(30) - i))
        cnt = jnp.sum(jnp.where(ki >= cand, 1, 0))
        return jnp.where(cnt >= K, cand, t)

    tstar = lax.fori_loop(0, 31, bs_step, jnp.int32(0))
    nmax = jnp.int32(0)
    for c in range(NCHAIN):
        n_c = jnp.sum(jnp.where(ki[RPC * c:RPC * (c + 1), :] >= tstar,
                                1, 0))
        nmax = jnp.maximum(nmax, n_c)

    def step(t, carry):
        ms, new = [], []
        for ch in range(NCHAIN):
            kc, cm = carry[ch]
            m = jnp.max(cm, axis=1, keepdims=True)
            kc = jnp.where(kc == m, KMIN, kc)
            cm = jnp.max(kc, axis=0, keepdims=True)
            ms.append(m)
            new.append((kc, cm))
        klist_ref[pl.ds(t, 1), :] = jnp.concatenate(ms, axis=1)
        return tuple(new)

    carry0 = tuple(
        (keys[RPC * c:RPC * (c + 1), :],
         jnp.max(keys[RPC * c:RPC * (c + 1), :], axis=0, keepdims=True))
        for c in range(NCHAIN))
    lax.fori_loop(0, nmax, step, carry0)

    kl = klist_ref[...]
    kli = lax.bitcast_convert_type(kl, jnp.int32)
    vals4 = lax.bitcast_convert_type(
        jnp.bitwise_and(kli, jnp.int32(~0x1FFF)), jnp.float32)
    idx4 = jnp.int32(8191) - jnp.bitwise_and(kli, jnp.int32(0x1FFF))
    klT = lax.transpose(kl, (1, 0))
    pos = lax.broadcasted_iota(jnp.int32, (K, 1), 0)
    lane512 = lax.broadcasted_iota(jnp.int32, (1, K), 1)
    merged = None
    for c in range(NCHAIN):
        colc = kl[:, c:c + 1]
        rank = pos
        for o in range(NCHAIN):
            if o == c:
                continue
            gt = klT[o:o + 1, :] > colc
            rank = rank + jnp.sum(jnp.where(gt, 1, 0), axis=1,
                                  keepdims=True)
        S_T = jnp.where(rank == lane512, 1.0, 0.0)
        cat = jnp.concatenate(
            [vals4[:, c:c + 1], idx4[:, c:c + 1].astype(jnp.float32)],
            axis=1)
        contrib = lax.dot_general(
            S_T, cat, (((0,), (0,)), ((), ())),
            precision=lax.Precision.HIGHEST,
            preferred_element_type=jnp.float32)
        merged = contrib if merged is None else merged + contrib
    vals = merged[:, 0:1]
    idxc = merged[:, 1:2].astype(jnp.int32)

    lane = lax.broadcasted_iota(jnp.int32, (1, G), 1)
    P = jnp.where(idxc == lane, 1.0, 0.0).astype(jnp.bfloat16)
    gid_emb = jnp.dot(P, emb_bf[...], preferred_element_type=jnp.float32)
    uvg = jnp.dot(P, uv[...].astype(jnp.bfloat16),
                  preferred_element_type=jnp.float32)

    mm = functools.partial(jnp.dot, precision=lax.Precision.HIGHEST,
                           preferred_element_type=jnp.float32)
    x = jnp.log(vals + EPS)
    h1 = _gelu(mm(x, eW1[...]) + eb1[...].reshape(1, D))
    e_emb = mm(h1, eW2[...]) + eb2[...].reshape(1, D)
    h2 = _gelu(mm(uvg, pW1[...]) + pb1[...].reshape(1, D))
    p_emb = mm(h2, pW2[...]) + pb2[...].reshape(1, D)
    out[pl.ds(1, K), :] = gid_emb + e_emb + p_emb

    g_pre = (mm(feat4[...], gW1[...][0:4, :])
             + hnorm2 * gW1[...][4:5, :]
             + gb1[...].reshape(1, D))
    out[pl.ds(0, 1), :] = mm(_gelu(g_pre), gW2[...]) + gb2[...].reshape(1, D)


def _tc_tokens(part2d, uv, emb_bf, feat4, *weights):
    return pl.pallas_call(
        _tc_body,
        out_shape=jax.ShapeDtypeStruct((K + 1, D), jnp.float32),
        scratch_shapes=[
            pltpu.VMEM((K, 4), jnp.float32),
        ],
    )(part2d, uv, emb_bf, feat4, *weights)


def kernel(h0, group_id, group_uv, snr_db, P, NrRF, NtRF, group_embed,
           eW1, eb1, eW2, eb2, pW1, pb1, pW2, pb2, gW1, gb1, gW2, gb2):
    part = _sc_energy(h0, group_id.astype(jnp.int32))
    part2d = part.reshape(NW * 64, 128)
    feat4 = jnp.stack([
        jnp.asarray(snr_db, jnp.float32),
        jnp.asarray(P, jnp.float32),
        jnp.asarray(NrRF, jnp.float32),
        jnp.asarray(NtRF, jnp.float32),
    ]).reshape(1, 4)
    return _tc_tokens(part2d, group_uv.astype(jnp.float32),
                      group_embed.astype(jnp.bfloat16), feat4,
                      eW1, eb1, eW2, eb2, pW1, pb1, pW2, pb2,
                      gW1, gb1, gW2, gb2)
